# baseline (speedup 1.0000x reference)
import functools

import jax
import jax.numpy as jnp
from jax import lax
from jax.experimental import pallas as pl
from jax.experimental.pallas import tpu as pltpu
from jax.experimental.pallas import tpu_sc as plsc

VOCAB = 1000000
EMBED_DIM = 64
BATCH = 4096
SEQ = 200

_NC = 2
_NS = 16
_NW = _NC * _NS

_B = BATCH * SEQ
_CHUNK = 128
_PER_W = _B // _NW
_NCHUNK = _PER_W // _CHUNK
_NBUF = 4


def _emb_kernel(idx_hbm, table_hbm, out_hbm, idx_v, rows_v, sems):
    wid = lax.axis_index("s") * _NC + lax.axis_index("c")
    chunk0 = wid * _NCHUNK
    row0 = wid * _PER_W

    pltpu.sync_copy(idx_hbm.at[pl.ds(chunk0, _NCHUNK)], idx_v)

    def start_gather(j, slot):
        pltpu.async_copy(table_hbm.at[idx_v.at[j]], rows_v.at[slot],
                         sems.at[slot])

    def wait_gather(j, slot):
        pltpu.make_async_copy(table_hbm.at[idx_v.at[j]], rows_v.at[slot],
                              sems.at[slot]).wait()

    def write_out(j, slot):
        pltpu.sync_copy(rows_v.at[slot],
                        out_hbm.at[pl.ds(row0 + j * _CHUNK, _CHUNK)])

    for b in range(_NBUF):
        start_gather(b, b)

    @pl.loop(0, _NCHUNK - _NBUF, step=_NBUF)
    def _(g):
        for b in range(_NBUF):
            j = g + b
            wait_gather(j, b)
            write_out(j, b)
            start_gather(j + _NBUF, b)

    for b in range(_NBUF):
        j = _NCHUNK - _NBUF + b
        wait_gather(j, b)
        write_out(j, b)


@functools.partial(jax.jit, donate_argnums=())
def _emb_lookup(idx2d, table):
    mesh = plsc.VectorSubcoreMesh(core_axis_name="c", subcore_axis_name="s")
    run = pl.kernel(
        _emb_kernel,
        out_type=jax.ShapeDtypeStruct((_B, EMBED_DIM), jnp.float32),
        mesh=mesh,
        scratch_types=[
            pltpu.VMEM((_NCHUNK, _CHUNK), jnp.int32),
            pltpu.VMEM((_NBUF, _CHUNK, EMBED_DIM), jnp.float32),
            pltpu.SemaphoreType.DMA((_NBUF,)),
        ],
        compiler_params=pltpu.CompilerParams(use_tc_tiling_on_sc=False),
    )
    return run(idx2d, table)


def kernel(inputs, table):
    idx2d = inputs.reshape(_B // _CHUNK, _CHUNK).astype(jnp.int32)
    out = _emb_lookup(idx2d, table)
    return out.reshape(BATCH, SEQ, EMBED_DIM)
